# Initial kernel scaffold; baseline (speedup 1.0000x reference)
import jax
import jax.numpy as jnp
from jax.experimental import pallas as pl


def kernel(texture_img_f, depth_img_f, original_scores, predicted_cdf):
    raise NotImplementedError("write your pallas kernel here")


# speedup vs baseline: 4.6518x; 4.6518x over previous
import functools

import jax
import jax.numpy as jnp
from jax import lax
from jax.experimental import pallas as pl
from jax.experimental.pallas import tpu as pltpu
from jax.experimental.pallas import tpu_sc as plsc

_B = 4096
_S = 200
_K = 9
_NW = 32
_RPW = _B // _NW
_NG = _RPW // 16
_TEMP = 0.07
_QW = 1.0 / 9.0
_AW = 0.08
_N2 = 2 * _B
_BM = 1024


def _sqrt16(a):
    i = plsc.bitcast(a, jnp.int32)
    y = plsc.bitcast(jnp.int32(0x5F3759DF) - (i >> 1), jnp.float32)
    for _ in range(4):
        y = y * (1.5 - 0.5 * a * y * y)
    return jnp.where(a > 0.0, a * y, 0.0)


@functools.cache
def _sc_cdf_emd_kernel():
    mesh = plsc.VectorSubcoreMesh(core_axis_name="c", subcore_axis_name="s")
    return pl.kernel(
        _sc_cdf_emd,
        mesh=mesh,
        out_type=(
            jax.ShapeDtypeStruct((_B * _K,), jnp.float32),
            jax.ShapeDtypeStruct((_NW * 16,), jnp.float32),
        ),
        scratch_types=[
            pltpu.VMEM((_RPW * _S,), jnp.float32),
            pltpu.VMEM((_RPW * _K,), jnp.float32),
            pltpu.VMEM((_RPW * _K,), jnp.float32),
            pltpu.VMEM((16,), jnp.float32),
        ],
        compiler_params=pltpu.CompilerParams(needs_layout_passes=False),
    )


def _sc_cdf_emd(scores_hbm, pred_hbm, cdf_out, emd_out, scores_v, pred_v,
                cdf_v, emd_v):
    wid = lax.axis_index("s") * 2 + lax.axis_index("c")
    base = wid * _RPW
    pltpu.sync_copy(scores_hbm.at[pl.ds(base * _S, _RPW * _S)], scores_v)
    pltpu.sync_copy(pred_hbm.at[pl.ds(base * _K, _RPW * _K)], pred_v)
    emd_acc = jnp.zeros((16,), jnp.float32)
    for g in range(_NG):
        rows = lax.iota(jnp.int32, 16) + (g * 16)
        row_s = rows * _S
        row_k = rows * _K

        def body(j, cnts, row_s=row_s):
            v = plsc.load_gather(scores_v, [row_s + j])
            return tuple(
                cnts[k] + jnp.where(v <= (k - 4.0), 1.0, 0.0)
                for k in range(_K))

        cnts = lax.fori_loop(
            0, _S, body, tuple(jnp.zeros((16,), jnp.float32)
                               for _ in range(_K)))
        sq = jnp.zeros((16,), jnp.float32)
        for k in range(_K):
            ck = cnts[k] * (1.0 / _S)
            plsc.store_scatter(cdf_v, [row_k + k], ck)
            pk = plsc.load_gather(pred_v, [row_k + k])
            d = pk - ck
            sq = sq + d * d
        emd_acc = emd_acc + _sqrt16(sq * (1.0 / _K))
    emd_v[...] = emd_acc
    pltpu.sync_copy(cdf_v, cdf_out.at[pl.ds(base * _K, _RPW * _K)])
    pltpu.sync_copy(emd_v, emd_out.at[pl.ds(wid * 16, 16)])


def _quantile16(y_v, rb9, theta, ymax):
    lo = jnp.zeros((16,), jnp.int32)
    hi = jnp.zeros((16,), jnp.int32) + _K
    for _ in range(4):
        mid = lo + ((hi - lo) >> 1)
        ym = plsc.load_gather(y_v, [rb9 + mid])
        go_left = theta <= ym
        lo = jnp.where(go_left, lo, mid)
        hi = jnp.where(go_left, mid, hi)
    idx = hi
    idx_mod = jnp.where(idx == 0, 1, idx)
    x_left = (idx_mod - 5).astype(jnp.float32)
    y_left = plsc.load_gather(y_v, [rb9 + jnp.maximum(idx - 1, 0)])
    col_s = jnp.clip(idx - 1, 0, _K - 2)
    s_lo = plsc.load_gather(y_v, [rb9 + col_s])
    s_hi = plsc.load_gather(y_v, [rb9 + col_s + 1])
    slope = s_hi - s_lo
    xn = x_left + (theta - y_left) / slope
    xn = jnp.where(theta < y_left, 0.0, xn)
    xn = jnp.where(theta > ymax, 4.0, xn)
    return jnp.where(slope == 0.0, x_left, xn)


@functools.cache
def _sc_quantile_kernel():
    mesh = plsc.VectorSubcoreMesh(core_axis_name="c", subcore_axis_name="s")
    return pl.kernel(
        _sc_quantile,
        mesh=mesh,
        out_type=jax.ShapeDtypeStruct((_NW * 16,), jnp.float32),
        scratch_types=[
            pltpu.VMEM((_B * _K,), jnp.float32),
            pltpu.VMEM((_B * _K,), jnp.float32),
            pltpu.VMEM((16,), jnp.float32),
        ],
        compiler_params=pltpu.CompilerParams(needs_layout_passes=False),
    )


def _sc_quantile(cdf_hbm, pred_hbm, q_out, cdf_v, pred_v, q_v):
    wid = lax.axis_index("s") * 2 + lax.axis_index("c")
    base = wid * _RPW
    pltpu.sync_copy(cdf_hbm, cdf_v)
    pltpu.sync_copy(pred_hbm, pred_v)

    def maxbody(i, m):
        return (jnp.maximum(m[0], cdf_v[pl.ds(i * 16, 16)]),
                jnp.maximum(m[1], pred_v[pl.ds(i * 16, 16)]))

    init = jnp.zeros((16,), jnp.float32) - 3.0e38
    mo, mp = lax.fori_loop(0, (_B * _K) // 16, maxbody, (init, init))
    ymax_o = lax.reduce_max(mo, axes=(0,))
    ymax_p = lax.reduce_max(mp, axes=(0,))
    qacc = jnp.zeros((16,), jnp.float32)
    for g in range(_NG):
        rb9 = (lax.iota(jnp.int32, 16) + (base + g * 16)) * _K
        for theta in (0.25, 0.5, 0.75):
            xo = _quantile16(cdf_v, rb9, theta, ymax_o)
            xp = _quantile16(pred_v, rb9, theta, ymax_p)
            qacc = qacc + jnp.abs(xp - xo)
    q_v[...] = qacc
    pltpu.sync_copy(q_v, q_out.at[pl.ds(wid * 16, 16)])


def _tc_ntxent_body(z_ref, out_ref, zn_ref):
    i = pl.program_id(0)

    @pl.when(i == 0)
    def _():
        z = z_ref[...]
        nrm = jnp.sqrt(jnp.sum(z * z, axis=1, keepdims=True)) + 1e-12
        zn_ref[...] = z / nrm

    zi = zn_ref[pl.ds(i * _BM, _BM), :]
    row_ids = lax.broadcasted_iota(jnp.int32, (_BM, _BM), 0) + i * _BM

    def body(j, acc):
        zj = zn_ref[pl.ds(j * _BM, _BM), :]
        s = lax.dot_general(zi, zj, (((1,), (1,)), ((), ())),
                            preferred_element_type=jnp.float32)
        s = s * (1.0 / _TEMP)
        col_ids = lax.broadcasted_iota(jnp.int32, (_BM, _BM), 1) + j * _BM
        es = jnp.where(row_ids == col_ids, 0.0, jnp.exp(s))
        return acc + jnp.sum(es, axis=1, keepdims=True)

    acc = lax.fori_loop(0, _N2 // _BM, body,
                        jnp.zeros((_BM, 1), jnp.float32))
    lse = jnp.log(acc)
    p = lax.rem(i * _BM + _B, _N2)
    zp = zn_ref[pl.ds(p, _BM), :]
    pos = jnp.sum(zi * zp, axis=1, keepdims=True) * (1.0 / _TEMP)
    out_ref[...] = jnp.full((1, 1, 128), jnp.sum(lse - pos), jnp.float32)


def _tc_ntxent(z):
    return pl.pallas_call(
        _tc_ntxent_body,
        grid=(_N2 // _BM,),
        in_specs=[pl.BlockSpec((_N2, 32), lambda i: (0, 0))],
        out_specs=pl.BlockSpec((1, 1, 128), lambda i: (i, 0, 0)),
        out_shape=jax.ShapeDtypeStruct((_N2 // _BM, 1, 128), jnp.float32),
        scratch_shapes=[pltpu.VMEM((_N2, 32), jnp.float32)],
    )(z)


def kernel(texture_img_f, depth_img_f, original_scores, predicted_cdf):
    pred1d = predicted_cdf.reshape(-1)
    cdf1d, emd_parts = _sc_cdf_emd_kernel()(original_scores.reshape(-1),
                                            pred1d)
    q_parts = _sc_quantile_kernel()(cdf1d, pred1d)
    z = jnp.concatenate([texture_img_f, depth_img_f], axis=0)
    lse_parts = _tc_ntxent(z)
    img = jnp.sum(lse_parts[:, 0, 0]) * (1.0 / _N2)
    return (jnp.sum(emd_parts) + jnp.sum(q_parts) * (_QW / 3.0)
            + img * _AW)


# speedup vs baseline: 5.9549x; 1.2801x over previous
import functools

import jax
import jax.numpy as jnp
from jax import lax
from jax.experimental import pallas as pl
from jax.experimental.pallas import tpu as pltpu
from jax.experimental.pallas import tpu_sc as plsc

_B = 4096
_S = 200
_K = 9
_NW = 32
_RPW = _B // _NW
_NG = _RPW // 16
_TEMP = 0.07
_QW = 1.0 / 9.0
_AW = 0.08
_N2 = 2 * _B
_BM = 1024


def _sqrt16(a):
    i = plsc.bitcast(a, jnp.int32)
    y = plsc.bitcast(jnp.int32(0x5F3759DF) - (i >> 1), jnp.float32)
    for _ in range(4):
        y = y * (1.5 - 0.5 * a * y * y)
    return jnp.where(a > 0.0, a * y, 0.0)


@functools.cache
def _sc_cdf_emd_kernel():
    mesh = plsc.VectorSubcoreMesh(core_axis_name="c", subcore_axis_name="s")
    return pl.kernel(
        _sc_cdf_emd,
        mesh=mesh,
        out_type=(
            jax.ShapeDtypeStruct((_B * _K,), jnp.float32),
            jax.ShapeDtypeStruct((_NW * 16,), jnp.float32),
            jax.ShapeDtypeStruct((_NW * 32,), jnp.float32),
        ),
        scratch_types=[
            pltpu.VMEM((_RPW * _S,), jnp.float32),
            pltpu.VMEM((_RPW * _K,), jnp.float32),
            pltpu.VMEM((_RPW * _K,), jnp.float32),
            pltpu.VMEM((16,), jnp.float32),
            pltpu.VMEM((32,), jnp.float32),
        ],
        compiler_params=pltpu.CompilerParams(needs_layout_passes=False),
    )


def _sc_cdf_emd(scores_hbm, pred_hbm, cdf_out, emd_out, max_out, scores_v,
                pred_v, cdf_v, emd_v, max_v):
    wid = lax.axis_index("s") * 2 + lax.axis_index("c")
    base = wid * _RPW
    pltpu.sync_copy(scores_hbm.at[pl.ds(base * _S, _RPW * _S)], scores_v)
    pltpu.sync_copy(pred_hbm.at[pl.ds(base * _K, _RPW * _K)], pred_v)
    emd_acc = jnp.zeros((16,), jnp.float32)
    cmax = jnp.zeros((16,), jnp.float32) - 3.0e38
    pmax = jnp.zeros((16,), jnp.float32) - 3.0e38
    for g in range(_NG):
        rows = lax.iota(jnp.int32, 16) + (g * 16)
        row_s = rows * _S
        row_k = rows * _K

        def body(j, cnts, row_s=row_s):
            v = plsc.load_gather(scores_v, [row_s + j])
            return tuple(
                cnts[k] + jnp.where(v <= (k - 4.0), 1.0, 0.0)
                for k in range(_K))

        cnts = lax.fori_loop(
            0, _S, body, tuple(jnp.zeros((16,), jnp.float32)
                               for _ in range(_K)))
        sq = jnp.zeros((16,), jnp.float32)
        for k in range(_K):
            ck = cnts[k] * (1.0 / _S)
            plsc.store_scatter(cdf_v, [row_k + k], ck)
            pk = plsc.load_gather(pred_v, [row_k + k])
            cmax = jnp.maximum(cmax, ck)
            pmax = jnp.maximum(pmax, pk)
            d = pk - ck
            sq = sq + d * d
        emd_acc = emd_acc + _sqrt16(sq * (1.0 / _K))
    emd_v[...] = emd_acc
    max_v[pl.ds(0, 16)] = cmax
    max_v[pl.ds(16, 16)] = pmax
    pltpu.sync_copy(cdf_v, cdf_out.at[pl.ds(base * _K, _RPW * _K)])
    pltpu.sync_copy(emd_v, emd_out.at[pl.ds(wid * 16, 16)])
    pltpu.sync_copy(max_v, max_out.at[pl.ds(wid * 32, 32)])


def _quantile16(y_v, rb9, theta, ymax):
    lo = jnp.zeros((16,), jnp.int32)
    hi = jnp.zeros((16,), jnp.int32) + _K
    for _ in range(4):
        mid = lo + ((hi - lo) >> 1)
        ym = plsc.load_gather(y_v, [rb9 + mid])
        go_left = theta <= ym
        lo = jnp.where(go_left, lo, mid)
        hi = jnp.where(go_left, mid, hi)
    idx = hi
    idx_mod = jnp.where(idx == 0, 1, idx)
    x_left = (idx_mod - 5).astype(jnp.float32)
    y_left = plsc.load_gather(y_v, [rb9 + jnp.maximum(idx - 1, 0)])
    col_s = jnp.clip(idx - 1, 0, _K - 2)
    s_lo = plsc.load_gather(y_v, [rb9 + col_s])
    s_hi = plsc.load_gather(y_v, [rb9 + col_s + 1])
    slope = s_hi - s_lo
    xn = x_left + (theta - y_left) / slope
    xn = jnp.where(theta < y_left, 0.0, xn)
    xn = jnp.where(theta > ymax, 4.0, xn)
    return jnp.where(slope == 0.0, x_left, xn)


@functools.cache
def _sc_quantile_kernel():
    mesh = plsc.VectorSubcoreMesh(core_axis_name="c", subcore_axis_name="s")
    return pl.kernel(
        _sc_quantile,
        mesh=mesh,
        out_type=jax.ShapeDtypeStruct((_NW * 16,), jnp.float32),
        scratch_types=[
            pltpu.VMEM((_RPW * _K,), jnp.float32),
            pltpu.VMEM((_RPW * _K,), jnp.float32),
            pltpu.VMEM((_NW * 32,), jnp.float32),
            pltpu.VMEM((16,), jnp.float32),
        ],
        compiler_params=pltpu.CompilerParams(needs_layout_passes=False),
    )


def _sc_quantile(cdf_hbm, pred_hbm, max_hbm, q_out, cdf_v, pred_v, max_v,
                 q_v):
    wid = lax.axis_index("s") * 2 + lax.axis_index("c")
    base = wid * _RPW
    pltpu.sync_copy(cdf_hbm.at[pl.ds(base * _K, _RPW * _K)], cdf_v)
    pltpu.sync_copy(pred_hbm.at[pl.ds(base * _K, _RPW * _K)], pred_v)
    pltpu.sync_copy(max_hbm, max_v)

    def maxbody(i, m):
        return (jnp.maximum(m[0], max_v[pl.ds(i * 32, 16)]),
                jnp.maximum(m[1], max_v[pl.ds(i * 32 + 16, 16)]))

    init = jnp.zeros((16,), jnp.float32) - 3.0e38
    mo, mp = lax.fori_loop(0, _NW, maxbody, (init, init))
    ymax_o = lax.reduce_max(mo, axes=(0,))
    ymax_p = lax.reduce_max(mp, axes=(0,))
    qacc = jnp.zeros((16,), jnp.float32)
    for g in range(_NG):
        rb9 = (lax.iota(jnp.int32, 16) + (g * 16)) * _K
        for theta in (0.25, 0.5, 0.75):
            xo = _quantile16(cdf_v, rb9, theta, ymax_o)
            xp = _quantile16(pred_v, rb9, theta, ymax_p)
            qacc = qacc + jnp.abs(xp - xo)
    q_v[...] = qacc
    pltpu.sync_copy(q_v, q_out.at[pl.ds(wid * 16, 16)])


def _tc_ntxent_body(z_ref, out_ref, zn_ref, accr_ref, accc_ref):
    i = pl.program_id(0)

    @pl.when(i == 0)
    def _():
        z = z_ref[...]
        nrm = jnp.sqrt(jnp.sum(z * z, axis=1, keepdims=True)) + 1e-12
        zn_ref[...] = z / nrm
        accr_ref[...] = jnp.zeros_like(accr_ref)
        accc_ref[...] = jnp.zeros_like(accc_ref)

    zi = zn_ref[pl.ds(i * _BM, _BM), :]

    def body(j, _):
        zj = zn_ref[pl.ds(j * _BM, _BM), :]
        s = lax.dot_general(zi, zj, (((1,), (1,)), ((), ())),
                            preferred_element_type=jnp.float32)
        e = jnp.exp(s * (1.0 / _TEMP))
        accr_ref[pl.ds(i * _BM, _BM), :] += jnp.sum(e, axis=1, keepdims=True)

        @pl.when(j > i)
        def _():
            accc_ref[pl.ds(j, 1), :] += jnp.sum(e, axis=0, keepdims=True)

        return 0

    lax.fori_loop(i, _N2 // _BM, body, 0)
    selfe = jnp.exp(jnp.sum(zi * zi, axis=1, keepdims=True) * (1.0 / _TEMP))
    colpart = accc_ref[pl.ds(i, 1), :].reshape(_BM, 1)
    lse = jnp.log(accr_ref[pl.ds(i * _BM, _BM), :] + colpart - selfe)
    p = lax.rem(i * _BM + _B, _N2)
    zp = zn_ref[pl.ds(p, _BM), :]
    pos = jnp.sum(zi * zp, axis=1, keepdims=True) * (1.0 / _TEMP)
    out_ref[...] = jnp.full((1, 1, 128), jnp.sum(lse - pos), jnp.float32)


def _tc_ntxent(z):
    return pl.pallas_call(
        _tc_ntxent_body,
        grid=(_N2 // _BM,),
        in_specs=[pl.BlockSpec((_N2, 32), lambda i: (0, 0))],
        out_specs=pl.BlockSpec((1, 1, 128), lambda i: (i, 0, 0)),
        out_shape=jax.ShapeDtypeStruct((_N2 // _BM, 1, 128), jnp.float32),
        scratch_shapes=[
            pltpu.VMEM((_N2, 32), jnp.float32),
            pltpu.VMEM((_N2, 1), jnp.float32),
            pltpu.VMEM((_N2 // _BM, _BM), jnp.float32),
        ],
    )(z)


def kernel(texture_img_f, depth_img_f, original_scores, predicted_cdf):
    pred1d = predicted_cdf.reshape(-1)
    cdf1d, emd_parts, max_parts = _sc_cdf_emd_kernel()(
        original_scores.reshape(-1), pred1d)
    q_parts = _sc_quantile_kernel()(cdf1d, pred1d, max_parts)
    z = jnp.concatenate([texture_img_f, depth_img_f], axis=0)
    lse_parts = _tc_ntxent(z)
    img = jnp.sum(lse_parts[:, 0, 0]) * (1.0 / _N2)
    return (jnp.sum(emd_parts) + jnp.sum(q_parts) * (_QW / 3.0)
            + img * _AW)
